# baseline (speedup 1.0000x reference)
import functools

import jax
import jax.numpy as jnp
from jax import lax
from jax.experimental import pallas as pl
from jax.experimental.pallas import tpu as pltpu
from jax.experimental.pallas import tpu_sc as plsc

N_ROWS = 100000
D = 128
S = 256
NC = 2
NS = 16
NW = NC * NS
CH = 128
FULL = (N_ROWS // (NW * CH)) * NW * CH
CHUNKS = FULL // (NW * CH)
EXTRA = (N_ROWS - FULL) // CH
TAIL = N_ROWS - FULL - EXTRA * CH
TAIL_OFF = FULL + EXTRA * CH


BIG = 256
NBUF = 3
NBIG = CHUNKS * CH // BIG


def _sc_body(feat_hbm, seg_hbm, psums_hbm, pcnts_hbm,
             acc, rows0, rows1, rows2, rows_e, idx_all, idx_e, idx32, cnt,
             zbuf, sem_g0, sem_g1, sem_g2, sem_s, sem_i, sem_e):
  c = lax.axis_index("c")
  s = lax.axis_index("s")
  w = s * NC + c
  base = w * (CHUNKS * CH)

  zv = jnp.zeros((16,), jnp.float32)
  ov = jnp.ones((16,), jnp.float32)
  for i in range(16):
    for k in range(D // 16):
      zbuf[i, pl.ds(k * 16, 16)] = zv
  for i in range(S // 16):
    cnt[pl.ds(i * 16, 16)] = zv

  sl = pl.ds(s * 16, 16)
  pltpu.sync_copy(zbuf, acc.at[sl])
  plsc.subcore_barrier()

  def count_ids(idx_vals):
    plsc.addupdate_scatter(cnt, [idx_vals], ov)

  bufs = (rows0, rows1, rows2)
  gsems = (sem_g0, sem_g1, sem_g2)

  def gather(b):
    return pltpu.async_copy(feat_hbm.at[pl.ds(base + b * BIG, BIG)],
                            bufs[b % NBUF], gsems[b % NBUF])

  gathers = [gather(0), gather(1)]
  eoff = FULL + w * CH

  @pl.when(w < EXTRA)
  def _():
    pltpu.async_copy(seg_hbm.at[pl.ds(eoff, CH)], idx_e, sem_e)
    pltpu.async_copy(feat_hbm.at[pl.ds(eoff, CH)], rows_e, sem_e)

  @pl.when(w == EXTRA)
  def _():
    pltpu.async_copy(seg_hbm.at[pl.ds(TAIL_OFF, TAIL)], idx32, sem_e)
    pltpu.async_copy(feat_hbm.at[pl.ds(TAIL_OFF, TAIL)],
                     rows_e.at[pl.ds(0, TAIL)], sem_e)
  idx_dmas = [
      pltpu.async_copy(seg_hbm.at[pl.ds(base + r * CH, CH)], idx_all.at[r],
                       sem_i)
      for r in range(CHUNKS)
  ]
  for d in idx_dmas:
    d.wait()

  scatters = {}
  for b in range(NBIG):
    for d in scatters.pop(b - 1, ()):
      d.wait()
    if b + 2 < NBIG:
      gathers.append(gather(b + 2))
    gathers[b].wait()
    buf = bufs[b % NBUF]
    scatters[b] = [
        pltpu.async_copy(buf.at[pl.ds(k * CH, CH)],
                         acc.at[idx_all.at[b * (BIG // CH) + k]],
                         sem_s, add=True)
        for k in range(BIG // CH)
    ]
    for k in range(BIG // CH):
      r = b * (BIG // CH) + k
      for q in range(CH // 16):
        count_ids(idx_all[r, pl.ds(q * 16, 16)])
  for ds_ in scatters.values():
    for d in ds_:
      d.wait()

  @pl.when(w < EXTRA)
  def _():
    pltpu.make_async_copy(seg_hbm.at[pl.ds(eoff, CH)], idx_e, sem_e).wait()
    pltpu.make_async_copy(feat_hbm.at[pl.ds(eoff, CH)], rows_e, sem_e).wait()
    pltpu.sync_copy(rows_e, acc.at[idx_e], add=True)
    for q in range(CH // 16):
      count_ids(idx_e[pl.ds(q * 16, 16)])

  @pl.when(w == EXTRA)
  def _():
    pltpu.make_async_copy(seg_hbm.at[pl.ds(TAIL_OFF, TAIL)], idx32,
                          sem_e).wait()
    pltpu.make_async_copy(feat_hbm.at[pl.ds(TAIL_OFF, TAIL)],
                          rows_e.at[pl.ds(0, TAIL)], sem_e).wait()
    pltpu.sync_copy(rows_e.at[pl.ds(0, TAIL)], acc.at[idx32], add=True)
    for q in range(TAIL // 16):
      count_ids(idx32[pl.ds(q * 16, 16)])

  plsc.subcore_barrier()

  pltpu.sync_copy(acc.at[sl], psums_hbm.at[c, sl])
  pltpu.sync_copy(cnt, pcnts_hbm.at[w])


_sc_pool = functools.partial(
    pl.kernel,
    out_type=(
        jax.ShapeDtypeStruct((NC, S, D), jnp.float32),
        jax.ShapeDtypeStruct((NW, S), jnp.float32),
    ),
    mesh=plsc.VectorSubcoreMesh(
        core_axis_name="c", subcore_axis_name="s",
        num_cores=NC, num_subcores=NS),
    scratch_types=[
        pltpu.VMEM_SHARED((S, D), jnp.float32),
        pltpu.VMEM((BIG, D), jnp.float32),
        pltpu.VMEM((BIG, D), jnp.float32),
        pltpu.VMEM((BIG, D), jnp.float32),
        pltpu.VMEM((CH, D), jnp.float32),
        pltpu.VMEM((CHUNKS, CH), jnp.int32),
        pltpu.VMEM((CH,), jnp.int32),
        pltpu.VMEM((TAIL,), jnp.int32),
        pltpu.VMEM((S,), jnp.float32),
        pltpu.VMEM((16, D), jnp.float32),
        pltpu.SemaphoreType.DMA,
        pltpu.SemaphoreType.DMA,
        pltpu.SemaphoreType.DMA,
        pltpu.SemaphoreType.DMA,
        pltpu.SemaphoreType.DMA,
        pltpu.SemaphoreType.DMA,
    ],
    compiler_params=pltpu.CompilerParams(needs_layout_passes=False),
)(_sc_body)


def _combine_body(ps_ref, pc_ref, o_ref):
  sums = ps_ref[0] + ps_ref[1]
  counts = jnp.sum(pc_ref[...], axis=0, keepdims=True)
  counts_col = jnp.transpose(counts)
  o_ref[...] = sums / jnp.clip(counts_col, 1.0, None)


def kernel(feat, segment_ids, num_graphs):
  seg = segment_ids.astype(jnp.int32)
  psums, pcnts = _sc_pool(feat, seg)
  out = pl.pallas_call(
      _combine_body,
      out_shape=jax.ShapeDtypeStruct((S, D), jnp.float32),
  )(psums, pcnts)
  return out
